# baseline (speedup 1.0000x reference)
import jax
import jax.numpy as jnp
from jax.experimental import pallas as pl
from jax.experimental.pallas import tpu as pltpu

B = 256
C = 256
HW = 256
NUM_FIXED = 51
P_MEM = 0.1
BBLK = 8


def _threefry2x32(k0, k1, x0, x1):
    ks2 = k0 ^ k1 ^ jnp.uint32(0x1BD11BDA)
    ks = [k0, k1, ks2]
    rotations = [(13, 15, 26, 6), (17, 29, 16, 24)]
    x0 = x0 + k0
    x1 = x1 + k1
    for i in range(5):
        for r in rotations[i % 2]:
            x0 = x0 + x1
            x1 = (x1 << jnp.uint32(r)) | (x1 >> jnp.uint32(32 - r))
            x1 = x1 ^ x0
        x0 = x0 + ks[(i + 1) % 3]
        x1 = x1 + ks[(i + 2) % 3] + jnp.uint32(i + 1)
    return x0, x1


def _mask_rows(idx_rows):
    i_u = jnp.broadcast_to(idx_rows.astype(jnp.uint32), (BBLK, C))
    zero = jnp.zeros((BBLK, C), jnp.uint32)
    k0b, k1b = _threefry2x32(jnp.uint32(0), jnp.uint32(42), zero, i_u)
    c = jax.lax.broadcasted_iota(jnp.uint32, (BBLK, C), 1)
    j = c - jnp.uint32(NUM_FIXED)
    o0, o1 = _threefry2x32(k0b, k1b, zero, j)
    bits = o0 ^ o1
    u = jax.lax.bitcast_convert_type(
        (bits >> jnp.uint32(9)) | jnp.uint32(0x3F800000), jnp.float32) - 1.0
    kept = (c < jnp.uint32(NUM_FIXED)) | (u < P_MEM)
    return kept.astype(jnp.float32)


def _body(idx_ref, x_ref, o_ref):
    p = pl.program_id(0)
    idx_rows = idx_ref[pl.ds(p * BBLK, BBLK), :]
    mask = _mask_rows(idx_rows)
    o_ref[...] = x_ref[...] * mask[:, :, None]


def kernel(X, idx):
    x3 = X.reshape(B, C, HW)
    out = pl.pallas_call(
        _body,
        grid=(B // BBLK,),
        in_specs=[
            pl.BlockSpec((B, 1), lambda i: (0, 0)),
            pl.BlockSpec((BBLK, C, HW), lambda i: (i, 0, 0)),
        ],
        out_specs=pl.BlockSpec((BBLK, C, HW), lambda i: (i, 0, 0)),
        out_shape=jax.ShapeDtypeStruct((B, C, HW), X.dtype),
    )(idx.reshape(B, 1), x3)
    return out.reshape(X.shape)


# speedup vs baseline: 3.3849x; 3.3849x over previous
import jax
import jax.numpy as jnp
from jax.experimental import pallas as pl
from jax.experimental.pallas import tpu as pltpu

B = 256
C = 256
HW = 256
NUM_FIXED = 51
P_MEM = 0.1
BBLK = 8


def _threefry2x32(k0, k1, x0, x1):
    ks2 = k0 ^ k1 ^ jnp.uint32(0x1BD11BDA)
    ks = [k0, k1, ks2]
    rotations = [(13, 15, 26, 6), (17, 29, 16, 24)]
    x0 = x0 + k0
    x1 = x1 + k1
    for i in range(5):
        for r in rotations[i % 2]:
            x0 = x0 + x1
            x1 = (x1 << jnp.uint32(r)) | (x1 >> jnp.uint32(32 - r))
            x1 = x1 ^ x0
        x0 = x0 + ks[(i + 1) % 3]
        x1 = x1 + ks[(i + 2) % 3] + jnp.uint32(i + 1)
    return x0, x1


def _mask_rows(idx_rows, nrows):
    i_u = jnp.broadcast_to(idx_rows.astype(jnp.uint32), (nrows, C))
    zero = jnp.zeros((nrows, C), jnp.uint32)
    k0b, k1b = _threefry2x32(jnp.uint32(0), jnp.uint32(42), zero, i_u)
    c = jax.lax.broadcasted_iota(jnp.uint32, (nrows, C), 1)
    j = c - jnp.uint32(NUM_FIXED)
    o0, o1 = _threefry2x32(k0b, k1b, zero, j)
    bits = o0 ^ o1
    u = jax.lax.bitcast_convert_type(
        (bits >> jnp.uint32(9)) | jnp.uint32(0x3F800000), jnp.float32) - 1.0
    kept = (c < jnp.uint32(NUM_FIXED)) | (u < P_MEM)
    return kept.astype(jnp.float32)


def _body(idx_ref, x_ref, o_ref):
    p = pl.program_id(0)
    idx_rows = idx_ref[pl.ds(p * BBLK, BBLK), :]
    mask = _mask_rows(idx_rows, BBLK)
    o_ref[...] = x_ref[...] * mask[:, None, :]


def kernel(X, idx):
    xt = jnp.transpose(X, (0, 2, 3, 1)).reshape(B, HW, C)
    out = pl.pallas_call(
        _body,
        grid=(B // BBLK,),
        in_specs=[
            pl.BlockSpec((B, 1), lambda i: (0, 0)),
            pl.BlockSpec((BBLK, HW, C), lambda i: (i, 0, 0)),
        ],
        out_specs=pl.BlockSpec((BBLK, HW, C), lambda i: (i, 0, 0)),
        out_shape=jax.ShapeDtypeStruct((B, HW, C), X.dtype),
    )(idx.reshape(B, 1), xt)
    return jnp.transpose(out.reshape(B, 16, 16, C), (0, 3, 1, 2))


# speedup vs baseline: 3.7844x; 1.1180x over previous
import jax
import jax.numpy as jnp
from jax.experimental import pallas as pl
from jax.experimental.pallas import tpu as pltpu

B = 256
C = 256
HW = 256
NUM_FIXED = 51
P_MEM = 0.1
BBLK = 16


def _threefry2x32(k0, k1, x0, x1):
    ks2 = k0 ^ k1 ^ jnp.uint32(0x1BD11BDA)
    ks = [k0, k1, ks2]
    rotations = [(13, 15, 26, 6), (17, 29, 16, 24)]
    x0 = x0 + k0
    x1 = x1 + k1
    for i in range(5):
        for r in rotations[i % 2]:
            x0 = x0 + x1
            x1 = (x1 << jnp.uint32(r)) | (x1 >> jnp.uint32(32 - r))
            x1 = x1 ^ x0
        x0 = x0 + ks[(i + 1) % 3]
        x1 = x1 + ks[(i + 2) % 3] + jnp.uint32(i + 1)
    return x0, x1


def _mask_rows(idx_rows, nrows):
    i_u = jnp.broadcast_to(idx_rows.astype(jnp.uint32), (nrows, C))
    zero = jnp.zeros((nrows, C), jnp.uint32)
    k0b, k1b = _threefry2x32(jnp.uint32(0), jnp.uint32(42), zero, i_u)
    c = jax.lax.broadcasted_iota(jnp.uint32, (nrows, C), 1)
    j = c - jnp.uint32(NUM_FIXED)
    o0, o1 = _threefry2x32(k0b, k1b, zero, j)
    bits = o0 ^ o1
    u = jax.lax.bitcast_convert_type(
        (bits >> jnp.uint32(9)) | jnp.uint32(0x3F800000), jnp.float32) - 1.0
    kept = (c < jnp.uint32(NUM_FIXED)) | (u < P_MEM)
    return kept.astype(jnp.float32)


def _body(idx_ref, x_ref, o_ref):
    p = pl.program_id(0)
    idx_rows = idx_ref[pl.ds(p * BBLK, BBLK), :]
    mask = _mask_rows(idx_rows, BBLK)
    o_ref[...] = x_ref[...] * mask[:, None, :]


def kernel(X, idx):
    xt = jnp.transpose(X, (0, 2, 3, 1)).reshape(B, HW, C)
    out = pl.pallas_call(
        _body,
        grid=(B // BBLK,),
        in_specs=[
            pl.BlockSpec((B, 1), lambda i: (0, 0)),
            pl.BlockSpec((BBLK, HW, C), lambda i: (i, 0, 0)),
        ],
        out_specs=pl.BlockSpec((BBLK, HW, C), lambda i: (i, 0, 0)),
        out_shape=jax.ShapeDtypeStruct((B, HW, C), X.dtype),
    )(idx.reshape(B, 1), xt)
    return jnp.transpose(out.reshape(B, 16, 16, C), (0, 3, 1, 2))


# speedup vs baseline: 3.9210x; 1.0361x over previous
import jax
import jax.numpy as jnp
from jax.experimental import pallas as pl
from jax.experimental.pallas import tpu as pltpu

B = 256
C = 256
HW = 256
NUM_FIXED = 51
P_MEM = 0.1
BBLK = 32


def _threefry2x32(k0, k1, x0, x1):
    ks2 = k0 ^ k1 ^ jnp.uint32(0x1BD11BDA)
    ks = [k0, k1, ks2]
    rotations = [(13, 15, 26, 6), (17, 29, 16, 24)]
    x0 = x0 + k0
    x1 = x1 + k1
    for i in range(5):
        for r in rotations[i % 2]:
            x0 = x0 + x1
            x1 = (x1 << jnp.uint32(r)) | (x1 >> jnp.uint32(32 - r))
            x1 = x1 ^ x0
        x0 = x0 + ks[(i + 1) % 3]
        x1 = x1 + ks[(i + 2) % 3] + jnp.uint32(i + 1)
    return x0, x1


def _mask_rows(idx_rows, nrows):
    i_u = jnp.broadcast_to(idx_rows.astype(jnp.uint32), (nrows, C))
    zero = jnp.zeros((nrows, C), jnp.uint32)
    k0b, k1b = _threefry2x32(jnp.uint32(0), jnp.uint32(42), zero, i_u)
    c = jax.lax.broadcasted_iota(jnp.uint32, (nrows, C), 1)
    j = c - jnp.uint32(NUM_FIXED)
    o0, o1 = _threefry2x32(k0b, k1b, zero, j)
    bits = o0 ^ o1
    u = jax.lax.bitcast_convert_type(
        (bits >> jnp.uint32(9)) | jnp.uint32(0x3F800000), jnp.float32) - 1.0
    kept = (c < jnp.uint32(NUM_FIXED)) | (u < P_MEM)
    return kept.astype(jnp.float32)


def _body(idx_ref, x_ref, o_ref):
    p = pl.program_id(0)
    idx_rows = idx_ref[pl.ds(p * BBLK, BBLK), :]
    mask = _mask_rows(idx_rows, BBLK)
    o_ref[...] = x_ref[...] * mask[:, None, :]


def kernel(X, idx):
    xt = jnp.transpose(X, (0, 2, 3, 1)).reshape(B, HW, C)
    out = pl.pallas_call(
        _body,
        grid=(B // BBLK,),
        in_specs=[
            pl.BlockSpec((B, 1), lambda i: (0, 0)),
            pl.BlockSpec((BBLK, HW, C), lambda i: (i, 0, 0)),
        ],
        out_specs=pl.BlockSpec((BBLK, HW, C), lambda i: (i, 0, 0)),
        out_shape=jax.ShapeDtypeStruct((B, HW, C), X.dtype),
    )(idx.reshape(B, 1), xt)
    return jnp.transpose(out.reshape(B, 16, 16, C), (0, 3, 1, 2))


# speedup vs baseline: 4.0923x; 1.0437x over previous
import jax
import jax.numpy as jnp
from jax.experimental import pallas as pl
from jax.experimental.pallas import tpu as pltpu

B = 256
C = 256
HW = 256
NUM_FIXED = 51
P_MEM = 0.1
BBLK = 32


def _threefry2x32(k0, k1, x0, x1):
    ks2 = k0 ^ k1 ^ jnp.uint32(0x1BD11BDA)
    ks = [k0, k1, ks2]
    rotations = [(13, 15, 26, 6), (17, 29, 16, 24)]
    x0 = x0 + k0
    x1 = x1 + k1
    for i in range(5):
        for r in rotations[i % 2]:
            x0 = x0 + x1
            x1 = (x1 << jnp.uint32(r)) | (x1 >> jnp.uint32(32 - r))
            x1 = x1 ^ x0
        x0 = x0 + ks[(i + 1) % 3]
        x1 = x1 + ks[(i + 2) % 3] + jnp.uint32(i + 1)
    return x0, x1


def _mask_rows(idx_rows, nrows):
    i_u = jnp.broadcast_to(idx_rows.astype(jnp.uint32), (nrows, C))
    zero = jnp.zeros((nrows, C), jnp.uint32)
    k0b, k1b = _threefry2x32(jnp.uint32(0), jnp.uint32(42), zero, i_u)
    c = jax.lax.broadcasted_iota(jnp.uint32, (nrows, C), 1)
    j = c - jnp.uint32(NUM_FIXED)
    o0, o1 = _threefry2x32(k0b, k1b, zero, j)
    bits = o0 ^ o1
    u = jax.lax.bitcast_convert_type(
        (bits >> jnp.uint32(9)) | jnp.uint32(0x3F800000), jnp.float32) - 1.0
    kept = (c < jnp.uint32(NUM_FIXED)) | (u < P_MEM)
    return kept.astype(jnp.float32)


def _body(idx_ref, x_ref, o_ref):
    p = pl.program_id(0)
    rows = [
        jnp.full((1, C), idx_ref[p * BBLK + r], jnp.int32) for r in range(BBLK)
    ]
    idx_rows = jnp.concatenate(rows, axis=0)
    mask = _mask_rows(idx_rows, BBLK)
    o_ref[...] = x_ref[...] * mask[:, None, :]


def kernel(X, idx):
    xt = jnp.transpose(X, (0, 2, 3, 1)).reshape(B, HW, C)
    out = pl.pallas_call(
        _body,
        grid=(B // BBLK,),
        in_specs=[
            pl.BlockSpec(memory_space=pltpu.MemorySpace.SMEM),
            pl.BlockSpec((BBLK, HW, C), lambda i: (i, 0, 0)),
        ],
        out_specs=pl.BlockSpec((BBLK, HW, C), lambda i: (i, 0, 0)),
        out_shape=jax.ShapeDtypeStruct((B, HW, C), X.dtype),
    )(idx, xt)
    return jnp.transpose(out.reshape(B, 16, 16, C), (0, 3, 1, 2))
